# baseline (speedup 1.0000x reference)
import functools

import jax
import jax.numpy as jnp
from jax import lax
from jax.experimental import pallas as pl
from jax.experimental.pallas import tpu as pltpu
from jax.experimental.pallas import tpu_sc as plsc

NC = 2
NS = 16
L = 16
NW = NC * NS

B = 16384
F = 128
ROWS_PER_W = B // NW
CHUNK = 128
NCHUNK = ROWS_PER_W // CHUNK
VIEW_COLS = CHUNK
VROWS_PER_W = ROWS_PER_W // VIEW_COLS


def _mf_body(user_hbm, item_hbm, ut_hbm, it_hbm, w_hbm, b_hbm, out_hbm,
             uidx, iidx, ubuf, vbuf, outv, wv, bv, sem):
    wid = lax.axis_index("s") * NC + lax.axis_index("c")
    base = wid * VROWS_PER_W

    pltpu.sync_copy(user_hbm.at[pl.ds(base, VROWS_PER_W)], uidx)
    pltpu.sync_copy(item_hbm.at[pl.ds(base, VROWS_PER_W)], iidx)
    pltpu.sync_copy(w_hbm, wv)
    pltpu.sync_copy(b_hbm, bv)

    lane = lax.broadcasted_iota(jnp.int32, (L,), 0)
    bias = bv[...]

    for j in range(NCHUNK):
        pltpu.async_copy(ut_hbm.at[uidx.at[j]], ubuf, sem).wait()
        pltpu.async_copy(it_hbm.at[iidx.at[j]], vbuf, sem).wait()
        for g in range(CHUNK // L):
            rows = lane + (g * L)

            def f_body(f, acc):
                cols = jnp.full((L,), f, dtype=jnp.int32)
                cu = plsc.load_gather(ubuf, [rows, cols])
                cv = plsc.load_gather(vbuf, [rows, cols])
                wf = plsc.load_gather(wv, [cols])
                return acc + cu * cv * wf

            acc = lax.fori_loop(0, F, f_body, bias)
            outv[j, pl.ds(g * L, L)] = acc

    pltpu.sync_copy(outv, out_hbm.at[pl.ds(base, VROWS_PER_W)])


@jax.jit
def _mf(user2d, item2d, user_table, item_table, w_flat, b16):
    kern = pl.kernel(
        _mf_body,
        out_type=jax.ShapeDtypeStruct((B // VIEW_COLS, VIEW_COLS), jnp.float32),
        mesh=plsc.VectorSubcoreMesh(
            core_axis_name="c", subcore_axis_name="s",
            num_cores=NC, num_subcores=NS),
        scratch_types=[
            pltpu.VMEM((VROWS_PER_W, VIEW_COLS), jnp.int32),
            pltpu.VMEM((VROWS_PER_W, VIEW_COLS), jnp.int32),
            pltpu.VMEM((CHUNK, F), jnp.float32),
            pltpu.VMEM((CHUNK, F), jnp.float32),
            pltpu.VMEM((VROWS_PER_W, VIEW_COLS), jnp.float32),
            pltpu.VMEM((F,), jnp.float32),
            pltpu.VMEM((L,), jnp.float32),
            pltpu.SemaphoreType.DMA,
        ],
        compiler_params=pltpu.CompilerParams(needs_layout_passes=False),
    )
    return kern(user2d, item2d, user_table, item_table, w_flat, b16)


def kernel(user, item, user_table, item_table, W, b):
    user2d = user.astype(jnp.int32).reshape(B // VIEW_COLS, VIEW_COLS)
    item2d = item.astype(jnp.int32).reshape(B // VIEW_COLS, VIEW_COLS)
    w_flat = W.reshape(F)
    b16 = jnp.broadcast_to(b.astype(jnp.float32), (L,))
    out = _mf(user2d, item2d, user_table, item_table, w_flat, b16)
    return out.reshape(-1)
